# baseline (speedup 1.0000x reference)
import functools

import jax
import jax.numpy as jnp
from jax import lax
from jax.experimental import pallas as pl
from jax.experimental.pallas import tpu as pltpu
from jax.experimental.pallas import tpu_sc as plsc

DATES, TIMES, B = 5000, 1000, 16384

_NC = 2
_NS = 16
_L = 16
_NW = _NC * _NS
_BPW = B // _NW
_CHUNKS = _BPW // _L
_GCH = _BPW // 128

_mesh = plsc.VectorSubcoreMesh(core_axis_name="c", subcore_axis_name="s")


@functools.partial(
    pl.kernel,
    out_type=jax.ShapeDtypeStruct((B,), jnp.float32),
    mesh=_mesh,
    scratch_types=[
        pltpu.VMEM((_BPW,), jnp.int32),
        pltpu.VMEM((_BPW,), jnp.int32),
        pltpu.VMEM((_BPW,), jnp.float32),
        pltpu.VMEM((_BPW,), jnp.float32),
        pltpu.VMEM((_BPW,), jnp.float32),
        pltpu.VMEM((_BPW,), jnp.int32),
        pltpu.VMEM((_BPW,), jnp.float32),
        pltpu.VMEM((_BPW,), jnp.float32),
        pltpu.SemaphoreType.DMA,
    ],
)
def _stop_loss_sc(d_hbm, t_hbm, pos_hbm, prev_hbm, act_hbm, table_hbm,
                  out_hbm, d_v, t_v, pos_v, prev_v, act_v, idx_v, g_v, o_v,
                  sem):
    wid = lax.axis_index("s") * _NC + lax.axis_index("c")
    base = wid * _BPW

    pltpu.sync_copy(d_hbm.at[pl.ds(base, _BPW)], d_v)
    pltpu.sync_copy(t_hbm.at[pl.ds(base, _BPW)], t_v)
    pltpu.sync_copy(pos_hbm.at[pl.ds(base, _BPW)], pos_v)
    pltpu.sync_copy(prev_hbm.at[pl.ds(base, _BPW)], prev_v)
    pltpu.sync_copy(act_hbm.at[pl.ds(base, _BPW)], act_v)

    def idx_body(i, _):
        s = pl.ds(pl.multiple_of(i * _L, _L), _L)
        pos = pos_v[s]
        act = act_v[s]
        t = t_v[s]
        p2 = jnp.where(pos == 0.0, jnp.int32(2000), jnp.int32(0))
        dirn = jnp.sign(pos + act)
        q = jnp.where(dirn > 0.0, jnp.int32(1000), jnp.int32(0))
        idx_v[s] = d_v[s] * 4000 + p2 + q + t
        return 0

    lax.fori_loop(0, _CHUNKS, idx_body, 0)

    copies = [
        pltpu.async_copy(table_hbm.at[idx_v.at[pl.ds(j * 128, 128)]],
                         g_v.at[pl.ds(j * 128, 128)], sem)
        for j in range(_GCH)
    ]
    for c in copies:
        c.wait()

    def out_body(i, _):
        s = pl.ds(pl.multiple_of(i * _L, _L), _L)
        pos = pos_v[s]
        act = act_v[s]
        pv = prev_v[s]
        g = g_v[s]
        dirn = jnp.sign(pos + act)
        prev_stop = jnp.where(
            pv != pv,
            jnp.where(dirn != 0.0, jnp.float32(-jnp.inf) * dirn, pv), pv)
        sp = jnp.where(dirn > 0.0, jnp.maximum(prev_stop, g),
                       jnp.minimum(prev_stop, g))
        o_v[s] = jnp.where(sp != sp, prev_stop,
                           jnp.where(dirn == 0.0, prev_stop, sp))
        return 0

    lax.fori_loop(0, _CHUNKS, out_body, 0)

    pltpu.sync_copy(o_v, out_hbm.at[pl.ds(base, _BPW)])


def kernel(date_idx, time_idx, position, prev_stop_loss, action,
           potential_stops):
    flat = jnp.transpose(potential_stops, (0, 2, 3, 1)).reshape(-1)
    return _stop_loss_sc(date_idx.astype(jnp.int32),
                         time_idx.astype(jnp.int32),
                         position, prev_stop_loss, action, flat)


# speedup vs baseline: 1.4565x; 1.4565x over previous
import functools

import jax
import jax.numpy as jnp
from jax import lax
from jax.experimental import pallas as pl
from jax.experimental.pallas import tpu as pltpu
from jax.experimental.pallas import tpu_sc as plsc

DATES, TIMES, B = 5000, 1000, 16384

_NC = 2
_NS = 16
_L = 16
_NW = _NC * _NS
_BPW = B // _NW
_CHUNKS = _BPW // _L
_GCH = _BPW // 128

_mesh = plsc.VectorSubcoreMesh(core_axis_name="c", subcore_axis_name="s")


@functools.partial(
    pl.kernel,
    out_type=jax.ShapeDtypeStruct((B,), jnp.float32),
    mesh=_mesh,
    scratch_types=[
        pltpu.VMEM((_BPW,), jnp.int32),
        pltpu.VMEM((_BPW,), jnp.int32),
        pltpu.VMEM((_BPW,), jnp.float32),
        pltpu.VMEM((_BPW,), jnp.float32),
        pltpu.VMEM((_BPW,), jnp.float32),
        pltpu.VMEM((_BPW,), jnp.int32),
        pltpu.VMEM((_BPW,), jnp.float32),
        pltpu.VMEM((_BPW,), jnp.float32),
        pltpu.SemaphoreType.DMA,
    ],
)
def _stop_loss_sc(d_hbm, t_hbm, pos_hbm, prev_hbm, act_hbm, table_hbm,
                  out_hbm, d_v, t_v, pos_v, prev_v, act_v, idx_v, g_v, o_v,
                  sem):
    wid = lax.axis_index("s") * _NC + lax.axis_index("c")
    base = wid * _BPW

    pltpu.sync_copy(d_hbm.at[pl.ds(base, _BPW)], d_v)
    pltpu.sync_copy(t_hbm.at[pl.ds(base, _BPW)], t_v)
    pltpu.sync_copy(pos_hbm.at[pl.ds(base, _BPW)], pos_v)
    pltpu.sync_copy(prev_hbm.at[pl.ds(base, _BPW)], prev_v)
    pltpu.sync_copy(act_hbm.at[pl.ds(base, _BPW)], act_v)

    def idx_body(i, _):
        s = pl.ds(pl.multiple_of(i * _L, _L), _L)
        pos = pos_v[s]
        act = act_v[s]
        t = t_v[s]
        p2 = jnp.where(pos == 0.0, jnp.int32(10000000), jnp.int32(0))
        dirn = jnp.sign(pos + act)
        q = jnp.where(dirn > 0.0, jnp.int32(5000000), jnp.int32(0))
        idx_v[s] = d_v[s] * 1000 + p2 + q + t
        return 0

    lax.fori_loop(0, _CHUNKS, idx_body, 0)

    copies = [
        pltpu.async_copy(table_hbm.at[idx_v.at[pl.ds(j * 128, 128)]],
                         g_v.at[pl.ds(j * 128, 128)], sem)
        for j in range(_GCH)
    ]
    for c in copies:
        c.wait()

    def out_body(i, _):
        s = pl.ds(pl.multiple_of(i * _L, _L), _L)
        pos = pos_v[s]
        act = act_v[s]
        pv = prev_v[s]
        g = g_v[s]
        dirn = jnp.sign(pos + act)
        prev_stop = jnp.where(
            pv != pv,
            jnp.where(dirn != 0.0, jnp.float32(-jnp.inf) * dirn, pv), pv)
        sp = jnp.where(dirn > 0.0, jnp.maximum(prev_stop, g),
                       jnp.minimum(prev_stop, g))
        o_v[s] = jnp.where(sp != sp, prev_stop,
                           jnp.where(dirn == 0.0, prev_stop, sp))
        return 0

    lax.fori_loop(0, _CHUNKS, out_body, 0)

    pltpu.sync_copy(o_v, out_hbm.at[pl.ds(base, _BPW)])


def kernel(date_idx, time_idx, position, prev_stop_loss, action,
           potential_stops):
    flat = jnp.transpose(potential_stops, (2, 3, 0, 1)).reshape(-1)
    return _stop_loss_sc(date_idx.astype(jnp.int32),
                         time_idx.astype(jnp.int32),
                         position, prev_stop_loss, action, flat)
